# baseline (speedup 1.0000x reference)
import functools
import math

import jax
import jax.numpy as jnp
from jax import lax
from jax.experimental import pallas as pl
from jax.experimental.pallas import tpu as pltpu
from jax.experimental.pallas import tpu_sc as plsc

_B, _S, _D = 4, 2048, 2048
_K = _S // 2
_NTOK = _B * _K
_FF = 4 * _D

_NC, _NS = 2, 16
_NW = _NC * _NS
_LANES = 16

_MSB_INT = -2147483648


def _scores_body(w_ref, x_ref, o_ref):
    s = lax.dot_general(
        w_ref[...], x_ref[...], (((1,), (1,)), ((), ())),
        preferred_element_type=jnp.float32)
    o_ref[...] = s[None]


def _scores_call(x2, router_w):
    ts = 1024
    nt = (_B * _S) // ts
    return pl.pallas_call(
        _scores_body,
        grid=(nt,),
        in_specs=[
            pl.BlockSpec((1, _D), lambda t: (0, 0)),
            pl.BlockSpec((ts, _D), lambda t: (t, 0)),
        ],
        out_specs=pl.BlockSpec((1, 1, ts), lambda t: (t, 0, 0)),
        out_shape=jax.ShapeDtypeStruct((nt, 1, ts), jnp.float32),
    )(router_w, x2)


def _route_body(scores_hbm, sel_idx_hbm, sel_s_hbm, unsel_idx_hbm,
                scores_v, keys_v, si_v, ss_v, ui_v):
    c = lax.axis_index("c")
    s = lax.axis_index("s")
    wid = s * _NC + c
    nchunk = _S // _LANES
    msb = jnp.int32(_MSB_INT)

    @pl.when(wid < _B)
    def _():
        b = wid
        pltpu.sync_copy(scores_hbm.at[b], scores_v)

        def mk(j, carry):
            sv = scores_v[pl.ds(j * _LANES, _LANES)]
            bits = lax.bitcast_convert_type(sv, jnp.int32)
            flip = jnp.right_shift(bits, 31) & jnp.int32(0x7FFFFFFF)
            keys_v[pl.ds(j * _LANES, _LANES)] = bits ^ flip
            return carry
        lax.fori_loop(0, nchunk, mk, jnp.int32(0))

        def outer(i, tu):
            bit = jnp.left_shift(jnp.int32(1), jnp.int32(31) - i)
            cand = tu | bit
            thr = cand ^ msb

            def cnt_body(j, acc):
                kv = keys_v[pl.ds(j * _LANES, _LANES)]
                return acc + plsc.all_reduce_population_count(kv >= thr)
            cnt_vec = lax.fori_loop(0, nchunk, cnt_body,
                                    jnp.zeros((_LANES,), jnp.int32))
            cnt = jnp.max(cnt_vec)
            return jnp.where(cnt >= _K, cand, tu)
        tu = lax.fori_loop(0, 32, outer, jnp.int32(0))
        thr_f = tu ^ msb

        def comp(j, carry):
            ps, pu = carry
            kv = keys_v[pl.ds(j * _LANES, _LANES)]
            sv = scores_v[pl.ds(j * _LANES, _LANES)]
            iv = lax.iota(jnp.int32, _LANES) + (j * _LANES + b * _S)
            m = kv >= thr_f
            cs = plsc.cumsum(m.astype(jnp.int32))
            pos_s = ps + cs - 1
            plsc.store_scatter(si_v, [pos_s], iv, mask=m)
            plsc.store_scatter(ss_v, [pos_s], sv, mask=m)
            nm = jnp.logical_not(m)
            cu = plsc.cumsum(nm.astype(jnp.int32))
            pos_u = pu + cu - 1
            plsc.store_scatter(ui_v, [pos_u], iv, mask=nm)
            cnt = jnp.max(cs)
            return ps + cnt, pu + (_LANES - cnt)
        lax.fori_loop(0, nchunk, comp, (jnp.int32(0), jnp.int32(0)))

        pltpu.sync_copy(si_v.at[pl.ds(0, _K)], sel_idx_hbm.at[b])
        pltpu.sync_copy(ss_v.at[pl.ds(0, _K)], sel_s_hbm.at[b])
        pltpu.sync_copy(ui_v.at[pl.ds(0, _K)], unsel_idx_hbm.at[b])


def _route_call(scores):
    mesh = plsc.VectorSubcoreMesh(core_axis_name="c", subcore_axis_name="s")
    pad = _K + _LANES
    f = pl.kernel(
        _route_body,
        mesh=mesh,
        compiler_params=pltpu.CompilerParams(needs_layout_passes=False),
        out_type=(
            jax.ShapeDtypeStruct((_B, _K), jnp.int32),
            jax.ShapeDtypeStruct((_B, _K), jnp.float32),
            jax.ShapeDtypeStruct((_B, _K), jnp.int32),
        ),
        scratch_types=[
            pltpu.VMEM((_S,), jnp.float32),
            pltpu.VMEM((_S,), jnp.int32),
            pltpu.VMEM((pad,), jnp.int32),
            pltpu.VMEM((pad,), jnp.float32),
            pltpu.VMEM((pad,), jnp.int32),
        ],
    )
    return f(scores)


_GCH = 16


def _gather_body(x_hbm, idx_hbm, out_hbm, idx_v, rows_v, sem):
    c = lax.axis_index("c")
    s = lax.axis_index("s")
    wid = s * _NC + c
    rows_per_w = _NTOK // _NW

    def body(t, carry):
        off = wid * rows_per_w + t * _GCH
        pltpu.sync_copy(idx_hbm.at[pl.ds(off, _GCH)], idx_v)
        pltpu.async_copy(x_hbm.at[idx_v], rows_v, sem).wait()
        pltpu.sync_copy(rows_v, out_hbm.at[pl.ds(off, _GCH)])
        return carry
    lax.fori_loop(0, rows_per_w // _GCH, body, jnp.int32(0))


def _gather_call(x2, sel_flat):
    mesh = plsc.VectorSubcoreMesh(core_axis_name="c", subcore_axis_name="s")
    f = pl.kernel(
        _gather_body,
        mesh=mesh,
        compiler_params=pltpu.CompilerParams(needs_layout_passes=False),
        out_type=jax.ShapeDtypeStruct((_NTOK, _D), jnp.float32),
        scratch_types=[
            pltpu.VMEM((_GCH,), jnp.int32),
            pltpu.VMEM((_GCH, _D), jnp.float32),
            pltpu.SemaphoreType.DMA,
        ],
    )
    return f(x2, sel_flat)


_BM = 512
_BF = 512


def _gelu_exact(h):
    return 0.5 * h * (1.0 + lax.erf(h * (1.0 / math.sqrt(2.0))))


def _mlp_body(a_ref, w1_ref, w2_ref, s_ref, o_ref):
    f = pl.program_id(1)
    nf = pl.num_programs(1)
    a_bf = a_ref[...].astype(jnp.bfloat16)
    w1_bf = w1_ref[...].astype(jnp.bfloat16)
    h = lax.dot_general(a_bf, w1_bf, (((1,), (1,)), ((), ())),
                        preferred_element_type=jnp.float32)
    g = _gelu_exact(h)
    w2_bf = w2_ref[...].astype(jnp.bfloat16)
    contrib = lax.dot_general(g.astype(jnp.bfloat16), w2_bf,
                              (((1,), (1,)), ((), ())),
                              preferred_element_type=jnp.float32)

    @pl.when(f == 0)
    def _():
        o_ref[...] = contrib

    @pl.when(f != 0)
    def _():
        o_ref[...] = o_ref[...] + contrib

    @pl.when(f == nf - 1)
    def _():
        gate = jax.nn.sigmoid(s_ref[...])
        o_ref[...] = o_ref[...] * gate


def _mlp_call(a, c_fc_w, c_proj_w, sel_s_col):
    grid = (_NTOK // _BM, _FF // _BF)
    return pl.pallas_call(
        _mlp_body,
        grid=grid,
        in_specs=[
            pl.BlockSpec((_BM, _D), lambda m, f: (m, 0)),
            pl.BlockSpec((_BF, _D), lambda m, f: (f, 0)),
            pl.BlockSpec((_D, _BF), lambda m, f: (0, f)),
            pl.BlockSpec((_BM, 1), lambda m, f: (m, 0)),
        ],
        out_specs=pl.BlockSpec((_BM, _D), lambda m, f: (m, 0)),
        out_shape=jax.ShapeDtypeStruct((_NTOK, _D), jnp.float32),
        compiler_params=pltpu.CompilerParams(
            dimension_semantics=("parallel", "arbitrary")),
    )(a, c_fc_w, c_proj_w, sel_s_col)


def _scatter_body(o_hbm, sidx_hbm, uidx_hbm, z_hbm, out_hbm,
                  idx_v, uidx_v, rows_v, zrows_v, sem):
    c = lax.axis_index("c")
    s = lax.axis_index("s")
    wid = s * _NC + c
    rows_per_w = _NTOK // _NW

    pltpu.sync_copy(z_hbm, zrows_v)

    def body(t, carry):
        off = wid * rows_per_w + t * _GCH
        pltpu.sync_copy(sidx_hbm.at[pl.ds(off, _GCH)], idx_v)
        pltpu.sync_copy(o_hbm.at[pl.ds(off, _GCH)], rows_v)
        pltpu.async_copy(rows_v, out_hbm.at[idx_v], sem).wait()
        pltpu.sync_copy(uidx_hbm.at[pl.ds(off, _GCH)], uidx_v)
        pltpu.async_copy(zrows_v, out_hbm.at[uidx_v], sem).wait()
        return carry
    lax.fori_loop(0, rows_per_w // _GCH, body, jnp.int32(0))


def _scatter_call(o, sel_flat, unsel_flat, zrows):
    mesh = plsc.VectorSubcoreMesh(core_axis_name="c", subcore_axis_name="s")
    f = pl.kernel(
        _scatter_body,
        mesh=mesh,
        compiler_params=pltpu.CompilerParams(needs_layout_passes=False),
        out_type=jax.ShapeDtypeStruct((_B * _S, _D), jnp.float32),
        scratch_types=[
            pltpu.VMEM((_GCH,), jnp.int32),
            pltpu.VMEM((_GCH,), jnp.int32),
            pltpu.VMEM((_GCH, _D), jnp.float32),
            pltpu.VMEM((_GCH, _D), jnp.float32),
            pltpu.SemaphoreType.DMA,
        ],
    )
    return f(o, sel_flat, unsel_flat, zrows)


def kernel(x, router_w, c_fc_w, c_proj_w):
    b, s, d = x.shape
    x2 = x.reshape(b * s, d)
    scores = _scores_call(x2, router_w).reshape(_B, _S)
    sel_idx, sel_s, unsel_idx = _route_call(scores)
    sel_flat = sel_idx.reshape(-1)
    unsel_flat = unsel_idx.reshape(-1)
    a = _gather_call(x2, sel_flat)
    o = _mlp_call(a, c_fc_w, c_proj_w, sel_s.reshape(-1, 1))
    zrows = jnp.zeros((_GCH, _D), jnp.float32)
    out2 = _scatter_call(o, sel_flat, unsel_flat, zrows)
    return out2.reshape(b, s, d)


# speedup vs baseline: 1.0124x; 1.0124x over previous
import functools
import math

import jax
import jax.numpy as jnp
from jax import lax
from jax.experimental import pallas as pl
from jax.experimental.pallas import tpu as pltpu
from jax.experimental.pallas import tpu_sc as plsc

_B, _S, _D = 4, 2048, 2048
_K = _S // 2
_NTOK = _B * _K
_FF = 4 * _D

_NC, _NS = 2, 16
_NW = _NC * _NS
_LANES = 16

_MSB_INT = -2147483648


def _scores_body(w_ref, x_ref, o_ref):
    s = lax.dot_general(
        w_ref[...], x_ref[...], (((1,), (1,)), ((), ())),
        preferred_element_type=jnp.float32)
    o_ref[...] = s[None]


def _scores_call(x2, router_w):
    ts = 1024
    nt = (_B * _S) // ts
    return pl.pallas_call(
        _scores_body,
        grid=(nt,),
        in_specs=[
            pl.BlockSpec((1, _D), lambda t: (0, 0)),
            pl.BlockSpec((ts, _D), lambda t: (t, 0)),
        ],
        out_specs=pl.BlockSpec((1, 1, ts), lambda t: (t, 0, 0)),
        out_shape=jax.ShapeDtypeStruct((nt, 1, ts), jnp.float32),
    )(router_w, x2)


def _route_body(scores_hbm, sel_idx_hbm, sel_s_hbm, unsel_idx_hbm,
                scores_v, keys_v, si_v, ss_v, ui_v):
    c = lax.axis_index("c")
    s = lax.axis_index("s")
    wid = s * _NC + c
    nchunk = _S // _LANES
    msb = jnp.int32(_MSB_INT)

    @pl.when(wid < _B)
    def _():
        b = wid
        pltpu.sync_copy(scores_hbm.at[b], scores_v)

        def mk(j, carry):
            sv = scores_v[pl.ds(j * _LANES, _LANES)]
            bits = lax.bitcast_convert_type(sv, jnp.int32)
            flip = jnp.right_shift(bits, 31) & jnp.int32(0x7FFFFFFF)
            keys_v[pl.ds(j * _LANES, _LANES)] = bits ^ flip
            return carry
        lax.fori_loop(0, nchunk, mk, jnp.int32(0))

        def outer(i, tu):
            bit = jnp.left_shift(jnp.int32(1), jnp.int32(31) - i)
            cand = tu | bit
            thr = cand ^ msb

            def cnt_body(j, acc):
                kv = keys_v[pl.ds(j * _LANES, _LANES)]
                return acc + plsc.all_reduce_population_count(kv >= thr)
            cnt_vec = lax.fori_loop(0, nchunk, cnt_body,
                                    jnp.zeros((_LANES,), jnp.int32))
            cnt = jnp.max(cnt_vec)
            return jnp.where(cnt >= _K, cand, tu)
        tu = lax.fori_loop(0, 32, outer, jnp.int32(0))
        thr_f = tu ^ msb

        def comp(j, carry):
            ps, pu = carry
            kv = keys_v[pl.ds(j * _LANES, _LANES)]
            sv = scores_v[pl.ds(j * _LANES, _LANES)]
            iv = lax.iota(jnp.int32, _LANES) + (j * _LANES + b * _S)
            m = kv >= thr_f
            cs = plsc.cumsum(m.astype(jnp.int32))
            pos_s = ps + cs - 1
            plsc.store_scatter(si_v, [pos_s], iv, mask=m)
            plsc.store_scatter(ss_v, [pos_s], sv, mask=m)
            nm = jnp.logical_not(m)
            cu = plsc.cumsum(nm.astype(jnp.int32))
            pos_u = pu + cu - 1
            plsc.store_scatter(ui_v, [pos_u], iv, mask=nm)
            cnt = jnp.max(cs)
            return ps + cnt, pu + (_LANES - cnt)
        lax.fori_loop(0, nchunk, comp, (jnp.int32(0), jnp.int32(0)))

        pltpu.sync_copy(si_v.at[pl.ds(0, _K)], sel_idx_hbm.at[b])
        pltpu.sync_copy(ss_v.at[pl.ds(0, _K)], sel_s_hbm.at[b])
        pltpu.sync_copy(ui_v.at[pl.ds(0, _K)], unsel_idx_hbm.at[b])


def _route_call(scores):
    mesh = plsc.VectorSubcoreMesh(core_axis_name="c", subcore_axis_name="s")
    pad = _K + _LANES
    f = pl.kernel(
        _route_body,
        mesh=mesh,
        compiler_params=pltpu.CompilerParams(needs_layout_passes=False),
        out_type=(
            jax.ShapeDtypeStruct((_B, _K), jnp.int32),
            jax.ShapeDtypeStruct((_B, _K), jnp.float32),
            jax.ShapeDtypeStruct((_B, _K), jnp.int32),
        ),
        scratch_types=[
            pltpu.VMEM((_S,), jnp.float32),
            pltpu.VMEM((_S,), jnp.int32),
            pltpu.VMEM((pad,), jnp.int32),
            pltpu.VMEM((pad,), jnp.float32),
            pltpu.VMEM((pad,), jnp.int32),
        ],
    )
    return f(scores)


_GCH = 16


def _gather_body(x_hbm, idx_hbm, out_hbm, idx_v, rows_v, sem):
    c = lax.axis_index("c")
    s = lax.axis_index("s")
    wid = s * _NC + c
    rows_per_w = _NTOK // _NW

    def body(t, carry):
        off = wid * rows_per_w + t * _GCH
        pltpu.sync_copy(idx_hbm.at[pl.ds(off, _GCH)], idx_v)
        pltpu.async_copy(x_hbm.at[idx_v], rows_v, sem).wait()
        pltpu.sync_copy(rows_v, out_hbm.at[pl.ds(off, _GCH)])
        return carry
    lax.fori_loop(0, rows_per_w // _GCH, body, jnp.int32(0))


def _gather_call(x2, sel_flat):
    mesh = plsc.VectorSubcoreMesh(core_axis_name="c", subcore_axis_name="s")
    f = pl.kernel(
        _gather_body,
        mesh=mesh,
        compiler_params=pltpu.CompilerParams(needs_layout_passes=False),
        out_type=jax.ShapeDtypeStruct((_NTOK, _D), jnp.float32),
        scratch_types=[
            pltpu.VMEM((_GCH,), jnp.int32),
            pltpu.VMEM((_GCH, _D), jnp.float32),
            pltpu.SemaphoreType.DMA,
        ],
    )
    return f(x2, sel_flat)


_BM = 1024
_BF = 512


def _gelu_exact(h):
    return 0.5 * h * (1.0 + lax.erf(h * (1.0 / math.sqrt(2.0))))


def _mlp_body(a_ref, w1_ref, w2_ref, s_ref, o_ref):
    f = pl.program_id(1)
    nf = pl.num_programs(1)
    a_bf = a_ref[...].astype(jnp.bfloat16)
    h = lax.dot_general(a_bf, w1_ref[...], (((1,), (1,)), ((), ())),
                        preferred_element_type=jnp.float32)
    g = _gelu_exact(h)
    contrib = lax.dot_general(g.astype(jnp.bfloat16), w2_ref[...],
                              (((1,), (1,)), ((), ())),
                              preferred_element_type=jnp.float32)

    @pl.when(f == 0)
    def _():
        o_ref[...] = contrib

    @pl.when(f != 0)
    def _():
        o_ref[...] = o_ref[...] + contrib

    @pl.when(f == nf - 1)
    def _():
        gate = jax.nn.sigmoid(s_ref[...])
        o_ref[...] = o_ref[...] * gate


def _mlp_call(a, c_fc_w, c_proj_w, sel_s_col):
    grid = (_NTOK // _BM, _FF // _BF)
    return pl.pallas_call(
        _mlp_body,
        grid=grid,
        in_specs=[
            pl.BlockSpec((_BM, _D), lambda m, f: (m, 0)),
            pl.BlockSpec((_BF, _D), lambda m, f: (f, 0)),
            pl.BlockSpec((_D, _BF), lambda m, f: (0, f)),
            pl.BlockSpec((_BM, 1), lambda m, f: (m, 0)),
        ],
        out_specs=pl.BlockSpec((_BM, _D), lambda m, f: (m, 0)),
        out_shape=jax.ShapeDtypeStruct((_NTOK, _D), jnp.float32),
        compiler_params=pltpu.CompilerParams(
            dimension_semantics=("parallel", "arbitrary")),
    )(a, c_fc_w, c_proj_w, sel_s_col)


def _scatter_body(o_hbm, sidx_hbm, uidx_hbm, z_hbm, out_hbm,
                  idx_v, uidx_v, rows_v, zrows_v, sem):
    c = lax.axis_index("c")
    s = lax.axis_index("s")
    wid = s * _NC + c
    rows_per_w = _NTOK // _NW

    pltpu.sync_copy(z_hbm, zrows_v)

    def body(t, carry):
        off = wid * rows_per_w + t * _GCH
        pltpu.sync_copy(sidx_hbm.at[pl.ds(off, _GCH)], idx_v)
        pltpu.sync_copy(o_hbm.at[pl.ds(off, _GCH)], rows_v)
        pltpu.async_copy(rows_v, out_hbm.at[idx_v], sem).wait()
        pltpu.sync_copy(uidx_hbm.at[pl.ds(off, _GCH)], uidx_v)
        pltpu.async_copy(zrows_v, out_hbm.at[uidx_v], sem).wait()
        return carry
    lax.fori_loop(0, rows_per_w // _GCH, body, jnp.int32(0))


def _scatter_call(o, sel_flat, unsel_flat, zrows):
    mesh = plsc.VectorSubcoreMesh(core_axis_name="c", subcore_axis_name="s")
    f = pl.kernel(
        _scatter_body,
        mesh=mesh,
        compiler_params=pltpu.CompilerParams(needs_layout_passes=False),
        out_type=jax.ShapeDtypeStruct((_B * _S, _D), jnp.float32),
        scratch_types=[
            pltpu.VMEM((_GCH,), jnp.int32),
            pltpu.VMEM((_GCH,), jnp.int32),
            pltpu.VMEM((_GCH, _D), jnp.float32),
            pltpu.VMEM((_GCH, _D), jnp.float32),
            pltpu.SemaphoreType.DMA,
        ],
    )
    return f(o, sel_flat, unsel_flat, zrows)


def kernel(x, router_w, c_fc_w, c_proj_w):
    b, s, d = x.shape
    x2 = x.reshape(b * s, d)
    scores = _scores_call(x2, router_w).reshape(_B, _S)
    sel_idx, sel_s, unsel_idx = _route_call(scores)
    sel_flat = sel_idx.reshape(-1)
    unsel_flat = unsel_idx.reshape(-1)
    a = _gather_call(x2, sel_flat)
    o = _mlp_call(a, c_fc_w.astype(jnp.bfloat16),
                  c_proj_w.astype(jnp.bfloat16), sel_s.reshape(-1, 1))
    zrows = jnp.zeros((_GCH, _D), jnp.float32)
    out2 = _scatter_call(o, sel_flat, unsel_flat, zrows)
    return out2.reshape(b, s, d)
